# baseline (speedup 1.0000x reference)
import functools

import jax
import jax.numpy as jnp
from jax import lax
from jax.experimental import pallas as pl
from jax.experimental.pallas import tpu as pltpu
from jax.experimental.pallas import tpu_sc as plsc

N_EMBD = 64
PADW = 128
SEQ = 200
BATCH = 4096
N_TOK = 100000
N_ROWS = BATCH * SEQ

_INFO = plsc.get_sparse_core_info()
NC, NS, L = _INFO.num_cores, _INFO.num_subcores, _INFO.num_lanes
NW = NC * NS

ROWS_PER_WORKER = N_ROWS // NW
CHUNK_ROWS = 128
CHUNKS = ROWS_PER_WORKER // CHUNK_ROWS
RING = 2

_mesh = plsc.VectorSubcoreMesh(core_axis_name="c", subcore_axis_name="s")


@functools.partial(
    pl.kernel,
    mesh=_mesh,
    out_type=jax.ShapeDtypeStruct((N_ROWS, N_EMBD), jnp.float32),
    scratch_types=[
        pltpu.VMEM((SEQ, N_EMBD), jnp.float32),
        pltpu.VMEM((RING, CHUNK_ROWS), jnp.int32),
        pltpu.VMEM((RING, CHUNK_ROWS, PADW), jnp.float32),
        pltpu.VMEM((RING, CHUNK_ROWS, N_EMBD), jnp.float32),
    ]
    + [pltpu.SemaphoreType.DMA] * RING
    + [pltpu.SemaphoreType.DMA] * RING,
)
def _emb_lookup(
    idx_hbm, tok_hbm, pos_hbm, out_hbm, pos_v, idx_v, rows_v, outs_v, *sems
):
    gsem = sems[:RING]
    osem = sems[RING:]
    wid = lax.axis_index("s") * NC + lax.axis_index("c")
    base_row = wid * ROWS_PER_WORKER
    pltpu.sync_copy(pos_hbm, pos_v)

    def issue_gather(h, slot):
        row0 = base_row + h * CHUNK_ROWS
        pltpu.sync_copy(idx_hbm.at[pl.ds(row0, CHUNK_ROWS)], idx_v.at[slot])
        pltpu.async_copy(tok_hbm.at[idx_v.at[slot]], rows_v.at[slot], gsem[slot])

    issue_gather(0, 0)

    def group_body(gg, carry):
        for b in range(RING):
            g = gg * RING + b

            @pl.when(g + 1 < CHUNKS)
            def _():
                issue_gather(g + 1, (b + 1) % RING)

            pltpu.make_async_copy(
                tok_hbm.at[idx_v.at[b]], rows_v.at[b], gsem[b]
            ).wait()

            @pl.when(g >= RING)
            def _():
                pltpu.make_async_copy(
                    outs_v.at[b],
                    out_hbm.at[pl.ds(base_row, CHUNK_ROWS)],
                    osem[b],
                ).wait()

            row0 = base_row + g * CHUNK_ROWS
            phase = lax.rem(row0, SEQ)

            def row_body(r, carry2):
                rp = phase + r
                rp = jnp.where(rp >= SEQ, rp - SEQ, rp)
                for c in range(N_EMBD // L):
                    col = pl.ds(c * L, L)
                    outs_v[b, r, col] = rows_v[b, r, col] + pos_v[rp, col]
                return carry2

            lax.fori_loop(0, CHUNK_ROWS, row_body, 0)

            pltpu.async_copy(
                outs_v.at[b],
                out_hbm.at[pl.ds(row0, CHUNK_ROWS)],
                osem[b],
            )
        return carry

    lax.fori_loop(0, CHUNKS // RING, group_body, 0)

    for b in range(RING):
        pltpu.make_async_copy(
            outs_v.at[b],
            out_hbm.at[pl.ds(base_row, CHUNK_ROWS)],
            osem[b],
        ).wait()


def kernel(x, tok_emb, pos_emb):
    idx = x.reshape(-1).astype(jnp.int32)
    tok_pad = jnp.pad(tok_emb, ((0, 0), (0, PADW - N_EMBD)))
    out = _emb_lookup(idx, tok_pad, pos_emb)
    return out.reshape(x.shape[0], x.shape[1], N_EMBD)
